# baseline (speedup 1.0000x reference)
import functools
import math

import jax
import jax.numpy as jnp
from jax import lax
from jax.experimental import pallas as pl
from jax.experimental.pallas import tpu as pltpu
from jax.experimental.pallas import tpu_sc as plsc

N = 10000
E = 320000
D = 128
BN_EPS = 1e-3

_NC = 2
_NS = 16
_NW = _NC * _NS
_CW = 128
_CHUNKS_PER_W = 80
_CHUNKS = _CHUNKS_PER_W * _NW
_EPAD = _CHUNKS * _CW
_RPT = 632
_NACC = _RPT * _NS
_GS = 8


_C0 = 32
_C1 = _CHUNKS_PER_W * 2 - _C0


@functools.lru_cache(maxsize=None)
def _sc_segment_sum(with_gather):
  mesh = plsc.VectorSubcoreMesh(core_axis_name="c", subcore_axis_name="s",
                                num_cores=_NC, num_subcores=_NS)
  scratch = [
      pltpu.VMEM((_GS, _CW), jnp.int32),
      pltpu.VMEM((_GS, _CW), jnp.int32),
      pltpu.VMEM((_CW,), jnp.int32),
      pltpu.VMEM((_CW,), jnp.int32),
      pltpu.VMEM((_CW,), jnp.int32),
      pltpu.VMEM((_CW,), jnp.int32),
      pltpu.VMEM((_CW, D), jnp.float32),
      pltpu.VMEM((_CW, D), jnp.float32),
      pltpu.VMEM_SHARED((_NACC, D), jnp.float32),
      pltpu.SemaphoreType.DMA,
      pltpu.SemaphoreType.DMA,
      pltpu.SemaphoreType.DMA,
      pltpu.SemaphoreType.DMA,
  ]

  def body(x_hbm, src_hbm, dst_hbm, out_hbm,
           src_v, dst_v, srcf0_v, dstf0_v, srcf1_v, dstf1_v,
           rows0_v, rows1_v, acc_sh, semg0, semg1, sems0, sems1):
    c = lax.axis_index("c")
    s = lax.axis_index("s")
    w = s * _NC + c
    r0 = s * _RPT
    cbase = w * _CHUNKS_PER_W
    rows_v = rows0_v

    fill = jnp.zeros((16,), jnp.float32)
    for r in range(_CW):
      for q in range(D // 16):
        rows_v[r, pl.ds(q * 16, 16)] = fill
    for t in range(_RPT // _CW + 1):
      rr = min(_CW, _RPT - t * _CW)
      pltpu.sync_copy(rows_v.at[pl.ds(0, rr)],
                      acc_sh.at[pl.ds(r0 + t * _CW, rr)])
    if not with_gather:
      one = jnp.ones((16,), jnp.float32)
      for r in range(_CW):
        for q in range(D // 16):
          rows_v[r, pl.ds(q * 16, 16)] = one

    plsc.subcore_barrier()

    if with_gather:
      bufs = ((srcf0_v, dstf0_v, rows0_v, semg0, sems0),
              (srcf1_v, dstf1_v, rows1_v, semg1, sems1))

      def _sub(j, b, base):
        srcf_b, dstf_b, rows_b, semg_b, sems_b = bufs[b]
        srcf_o, dstf_o, rows_o, semg_o, sems_o = bufs[1 - b]

        @pl.when(j % _GS == 0)
        def _stage():
          pltpu.sync_copy(src_hbm.at[pl.ds(base + (j // _GS) * _GS, _GS)],
                          src_v)
          pltpu.sync_copy(dst_hbm.at[pl.ds(base + (j // _GS) * _GS, _GS)],
                          dst_v)

        @pl.when(j >= 2)
        def _wait_prev_scatter():
          pltpu.make_async_copy(rows_b, acc_sh.at[dstf_b], sems_b).wait()

        jm = j % _GS
        for r in range(_CW // 16):
          srcf_b[pl.ds(r * 16, 16)] = src_v[jm, pl.ds(r * 16, 16)]
          dstf_b[pl.ds(r * 16, 16)] = dst_v[jm, pl.ds(r * 16, 16)]
        pltpu.async_copy(x_hbm.at[srcf_b], rows_b, semg_b)

        @pl.when(j >= 1)
        def _start_prev_scatter():
          pltpu.make_async_copy(x_hbm.at[srcf_o], rows_o, semg_o).wait()
          pltpu.async_copy(rows_o, acc_sh.at[dstf_o], sems_o, add=True)

      def _pipe(base, count):
        @pl.loop(0, count // 2)
        def _pair(p):
          _sub(2 * p, 0, base)
          _sub(2 * p + 1, 1, base)

        pltpu.make_async_copy(x_hbm.at[srcf1_v], rows1_v, semg1).wait()
        pltpu.async_copy(rows1_v, acc_sh.at[dstf1_v], sems1, add=True)
        pltpu.make_async_copy(rows0_v, acc_sh.at[dstf0_v], sems0).wait()
        pltpu.make_async_copy(rows1_v, acc_sh.at[dstf1_v], sems1).wait()

      @pl.when(c == 0)
      def _sc0():
        _pipe(s * _C0, _C0)

      @pl.when(c == 1)
      def _sc1():
        _pipe(_NS * _C0 + s * _C1, _C1)
    else:
      @pl.loop(0, _CHUNKS_PER_W // _GS)
      def _group(g):
        pltpu.sync_copy(dst_hbm.at[pl.ds(cbase + g * _GS, _GS)], dst_v)

        @pl.loop(0, _GS)
        def _chunk(j):
          for r in range(_CW // 16):
            dstf0_v[pl.ds(r * 16, 16)] = dst_v[j, pl.ds(r * 16, 16)]
          pltpu.sync_copy(rows_v, acc_sh.at[dstf0_v], add=True)

    plsc.subcore_barrier()

    for t in range(_RPT // _CW + 1):
      rr = min(_CW, _RPT - t * _CW)
      pltpu.sync_copy(acc_sh.at[pl.ds(r0 + t * _CW, rr)],
                      rows_v.at[pl.ds(0, rr)])
      pltpu.sync_copy(rows_v.at[pl.ds(0, rr)],
                      out_hbm.at[c, pl.ds(r0 + t * _CW, rr)])

  return pl.kernel(
      body,
      out_type=jax.ShapeDtypeStruct((_NC, _NACC, D), jnp.float32),
      mesh=mesh, scratch_types=scratch)


_BN_ROWS = 1000
_GRID = N // _BN_ROWS
_BN_SCALE = 1.0 / math.sqrt(1.0 + BN_EPS)


def _sage_block(xb, pb, dpb, w_ref, b_ref, g_ref, be_ref):
  psum = pb[0] + pb[1]
  deg = dpb[0, :, 0] + dpb[1, :, 0]
  agg = psum * (1.0 / jnp.maximum(deg, 1.0))[:, None]
  w = w_ref[...]
  h = (jnp.dot(xb, w[:D], preferred_element_type=jnp.float32)
       + jnp.dot(agg, w[D:], preferred_element_type=jnp.float32)
       + b_ref[...])
  nrm = jnp.sqrt(jnp.maximum(jnp.sum(h * h, axis=-1, keepdims=True), 1e-12))
  h = jnp.maximum(h / nrm, 0.0)
  return (g_ref[...] * _BN_SCALE) * h + be_ref[...]


def _layer_body(x_ref, p_ref, dp_ref, w_ref, b_ref, g_ref, be_ref, o_ref):
  o_ref[...] = _sage_block(x_ref[...], p_ref[...], dp_ref, w_ref, b_ref,
                           g_ref, be_ref)


def _final_body(x2_ref, p_ref, dp_ref, w_ref, b_ref, g_ref, be_ref,
                x1_ref, wl_ref, bl_ref, o_ref):
  x2b = x2_ref[...]
  x3b = _sage_block(x2b, p_ref[...], dp_ref, w_ref, b_ref, g_ref, be_ref)
  wl = wl_ref[...]
  x1b = x1_ref[...]
  out = (jnp.dot(x1b, wl[:D], preferred_element_type=jnp.float32)
         + jnp.dot(x2b, wl[D:2 * D], preferred_element_type=jnp.float32)
         + jnp.dot(x3b, wl[2 * D:], preferred_element_type=jnp.float32)
         + bl_ref[...])
  o_ref[...] = jnp.maximum(out, 0.0)


_row_spec = pl.BlockSpec((_BN_ROWS, D), lambda i: (i, 0))
_p_spec = pl.BlockSpec((_NC, _BN_ROWS, D), lambda i: (0, i, 0))
_w_spec = pl.BlockSpec((2 * D, D), lambda i: (0, 0))
_v_spec = pl.BlockSpec((1, D), lambda i: (0, 0))

_layer_call = pl.pallas_call(
    _layer_body,
    grid=(_GRID,),
    in_specs=[_row_spec, _p_spec, _p_spec, _w_spec, _v_spec, _v_spec,
              _v_spec],
    out_specs=_row_spec,
    out_shape=jax.ShapeDtypeStruct((N, D), jnp.float32),
)

_final_call = pl.pallas_call(
    _final_body,
    grid=(_GRID,),
    in_specs=[_row_spec, _p_spec, _p_spec, _w_spec, _v_spec, _v_spec,
              _v_spec, _row_spec, pl.BlockSpec((3 * D, D), lambda i: (0, 0)),
              _v_spec],
    out_specs=_row_spec,
    out_shape=jax.ShapeDtypeStruct((N, D), jnp.float32),
)


def kernel(x, edge_index, W1, b1, g1, be1, W2, b2, g2, be2, W3, b3, g3, be3,
           Wl, bl):
  src = edge_index[0]
  dst = edge_index[1]
  npad = _EPAD - E
  src_p = jnp.concatenate(
      [src, jnp.zeros((npad,), jnp.int32)]).reshape(_CHUNKS, _CW)
  dst_p = jnp.concatenate(
      [dst, N + (jnp.arange(npad, dtype=jnp.int32) % 16)]).reshape(
          _CHUNKS, _CW)

  seg = _sc_segment_sum(True)
  row = lambda v: v.reshape(1, D)
  dp = _sc_segment_sum(False)(x, src_p, dst_p)
  p1 = seg(x, src_p, dst_p)
  x1 = _layer_call(x, p1, dp, W1, row(b1), row(g1), row(be1))
  p2 = seg(x1, src_p, dst_p)
  x2 = _layer_call(x1, p2, dp, W2, row(b2), row(g2), row(be2))
  p3 = seg(x2, src_p, dst_p)
  out = _final_call(x2, p3, dp, W3, row(b3), row(g3), row(be3), x1, Wl,
                    bl.reshape(1, D))
  return out


# speedup vs baseline: 1.3624x; 1.3624x over previous
import functools
import math

import jax
import jax.numpy as jnp
from jax import lax
from jax.experimental import pallas as pl
from jax.experimental.pallas import tpu as pltpu
from jax.experimental.pallas import tpu_sc as plsc

N = 10000
E = 320000
D = 128
BN_EPS = 1e-3

_NC = 2
_NS = 16
_NW = _NC * _NS
_CW = 128
_CHUNKS_PER_W = 80
_CHUNKS = _CHUNKS_PER_W * _NW
_EPAD = _CHUNKS * _CW
_RPT = 632
_NACC = _RPT * _NS
_GS = 8


_C0 = 152
_C1 = _CHUNKS_PER_W * 2 - _C0


@functools.lru_cache(maxsize=None)
def _sc_segment_sum(with_gather):
  mesh = plsc.VectorSubcoreMesh(core_axis_name="c", subcore_axis_name="s",
                                num_cores=_NC, num_subcores=_NS)
  scratch = [
      pltpu.VMEM((_GS, _CW), jnp.int32),
      pltpu.VMEM((_GS, _CW), jnp.int32),
      pltpu.VMEM((_CW,), jnp.int32),
      pltpu.VMEM((_CW,), jnp.int32),
      pltpu.VMEM((_CW,), jnp.int32),
      pltpu.VMEM((_CW,), jnp.int32),
      pltpu.VMEM((_CW, D), jnp.float32),
      pltpu.VMEM((_CW, D), jnp.float32),
      pltpu.VMEM_SHARED((_NACC, D), jnp.float32),
      pltpu.SemaphoreType.DMA,
      pltpu.SemaphoreType.DMA,
      pltpu.SemaphoreType.DMA,
      pltpu.SemaphoreType.DMA,
  ]

  def body(x_hbm, src_hbm, dst_hbm, out_hbm,
           src_v, dst_v, srcf0_v, dstf0_v, srcf1_v, dstf1_v,
           rows0_v, rows1_v, acc_sh, semg0, semg1, sems0, sems1):
    c = lax.axis_index("c")
    s = lax.axis_index("s")
    w = s * _NC + c
    r0 = s * _RPT
    cbase = w * _CHUNKS_PER_W
    rows_v = rows0_v

    fill = jnp.zeros((16,), jnp.float32)
    for r in range(_CW):
      for q in range(D // 16):
        rows_v[r, pl.ds(q * 16, 16)] = fill
    for t in range(_RPT // _CW + 1):
      rr = min(_CW, _RPT - t * _CW)
      pltpu.sync_copy(rows_v.at[pl.ds(0, rr)],
                      acc_sh.at[pl.ds(r0 + t * _CW, rr)])
    if not with_gather:
      one = jnp.ones((16,), jnp.float32)
      for r in range(_CW):
        for q in range(D // 16):
          rows_v[r, pl.ds(q * 16, 16)] = one

    plsc.subcore_barrier()

    if with_gather:
      bufs = ((srcf0_v, dstf0_v, rows0_v, semg0, sems0),
              (srcf1_v, dstf1_v, rows1_v, semg1, sems1))

      def _sub(j, b, base):
        srcf_b, dstf_b, rows_b, semg_b, sems_b = bufs[b]
        srcf_o, dstf_o, rows_o, semg_o, sems_o = bufs[1 - b]

        @pl.when(j % _GS == 0)
        def _stage():
          pltpu.sync_copy(src_hbm.at[pl.ds(base + (j // _GS) * _GS, _GS)],
                          src_v)
          pltpu.sync_copy(dst_hbm.at[pl.ds(base + (j // _GS) * _GS, _GS)],
                          dst_v)

        @pl.when(j >= 2)
        def _wait_prev_scatter():
          pltpu.make_async_copy(rows_b, acc_sh.at[dstf_b], sems_b).wait()

        jm = j % _GS
        for r in range(_CW // 16):
          srcf_b[pl.ds(r * 16, 16)] = src_v[jm, pl.ds(r * 16, 16)]
          dstf_b[pl.ds(r * 16, 16)] = dst_v[jm, pl.ds(r * 16, 16)]
        pltpu.async_copy(x_hbm.at[srcf_b], rows_b, semg_b)

        @pl.when(j >= 1)
        def _start_prev_scatter():
          pltpu.make_async_copy(x_hbm.at[srcf_o], rows_o, semg_o).wait()
          pltpu.async_copy(rows_o, acc_sh.at[dstf_o], sems_o, add=True)

      def _pipe(base, count):
        @pl.loop(0, count // 2)
        def _pair(p):
          _sub(2 * p, 0, base)
          _sub(2 * p + 1, 1, base)

        pltpu.make_async_copy(x_hbm.at[srcf1_v], rows1_v, semg1).wait()
        pltpu.async_copy(rows1_v, acc_sh.at[dstf1_v], sems1, add=True)
        pltpu.make_async_copy(rows0_v, acc_sh.at[dstf0_v], sems0).wait()
        pltpu.make_async_copy(rows1_v, acc_sh.at[dstf1_v], sems1).wait()

      @pl.when(c == 0)
      def _sc0():
        _pipe(s * _C0, _C0)

      @pl.when(c == 1)
      def _sc1():
        _pipe(_NS * _C0 + s * _C1, _C1)
    else:
      @pl.loop(0, _CHUNKS_PER_W // _GS)
      def _group(g):
        pltpu.sync_copy(dst_hbm.at[pl.ds(cbase + g * _GS, _GS)], dst_v)

        @pl.loop(0, _GS)
        def _chunk(j):
          for r in range(_CW // 16):
            dstf0_v[pl.ds(r * 16, 16)] = dst_v[j, pl.ds(r * 16, 16)]
          pltpu.sync_copy(rows_v, acc_sh.at[dstf0_v], add=True)

    plsc.subcore_barrier()

    for t in range(_RPT // _CW + 1):
      rr = min(_CW, _RPT - t * _CW)
      pltpu.sync_copy(acc_sh.at[pl.ds(r0 + t * _CW, rr)],
                      rows_v.at[pl.ds(0, rr)])
      pltpu.sync_copy(rows_v.at[pl.ds(0, rr)],
                      out_hbm.at[c, pl.ds(r0 + t * _CW, rr)])

  return pl.kernel(
      body,
      out_type=jax.ShapeDtypeStruct((_NC, _NACC, D), jnp.float32),
      mesh=mesh, scratch_types=scratch)


_BN_ROWS = 1000
_GRID = N // _BN_ROWS
_BN_SCALE = 1.0 / math.sqrt(1.0 + BN_EPS)


def _sage_block(xb, pb, dpb, w_ref, b_ref, g_ref, be_ref):
  psum = pb[0] + pb[1]
  deg = dpb[0, :, 0] + dpb[1, :, 0]
  agg = psum * (1.0 / jnp.maximum(deg, 1.0))[:, None]
  w = w_ref[...]
  h = (jnp.dot(xb, w[:D], preferred_element_type=jnp.float32)
       + jnp.dot(agg, w[D:], preferred_element_type=jnp.float32)
       + b_ref[...])
  nrm = jnp.sqrt(jnp.maximum(jnp.sum(h * h, axis=-1, keepdims=True), 1e-12))
  h = jnp.maximum(h / nrm, 0.0)
  return (g_ref[...] * _BN_SCALE) * h + be_ref[...]


def _layer_body(x_ref, p_ref, dp_ref, w_ref, b_ref, g_ref, be_ref, o_ref):
  o_ref[...] = _sage_block(x_ref[...], p_ref[...], dp_ref, w_ref, b_ref,
                           g_ref, be_ref)


def _final_body(x2_ref, p_ref, dp_ref, w_ref, b_ref, g_ref, be_ref,
                x1_ref, wl_ref, bl_ref, o_ref):
  x2b = x2_ref[...]
  x3b = _sage_block(x2b, p_ref[...], dp_ref, w_ref, b_ref, g_ref, be_ref)
  wl = wl_ref[...]
  x1b = x1_ref[...]
  out = (jnp.dot(x1b, wl[:D], preferred_element_type=jnp.float32)
         + jnp.dot(x2b, wl[D:2 * D], preferred_element_type=jnp.float32)
         + jnp.dot(x3b, wl[2 * D:], preferred_element_type=jnp.float32)
         + bl_ref[...])
  o_ref[...] = jnp.maximum(out, 0.0)


_row_spec = pl.BlockSpec((_BN_ROWS, D), lambda i: (i, 0))
_p_spec = pl.BlockSpec((_NC, _BN_ROWS, D), lambda i: (0, i, 0))
_w_spec = pl.BlockSpec((2 * D, D), lambda i: (0, 0))
_v_spec = pl.BlockSpec((1, D), lambda i: (0, 0))

_layer_call = pl.pallas_call(
    _layer_body,
    grid=(_GRID,),
    in_specs=[_row_spec, _p_spec, _p_spec, _w_spec, _v_spec, _v_spec,
              _v_spec],
    out_specs=_row_spec,
    out_shape=jax.ShapeDtypeStruct((N, D), jnp.float32),
)

_final_call = pl.pallas_call(
    _final_body,
    grid=(_GRID,),
    in_specs=[_row_spec, _p_spec, _p_spec, _w_spec, _v_spec, _v_spec,
              _v_spec, _row_spec, pl.BlockSpec((3 * D, D), lambda i: (0, 0)),
              _v_spec],
    out_specs=_row_spec,
    out_shape=jax.ShapeDtypeStruct((N, D), jnp.float32),
)


def kernel(x, edge_index, W1, b1, g1, be1, W2, b2, g2, be2, W3, b3, g3, be3,
           Wl, bl):
  src = edge_index[0]
  dst = edge_index[1]
  npad = _EPAD - E
  src_p = jnp.concatenate(
      [src, jnp.zeros((npad,), jnp.int32)]).reshape(_CHUNKS, _CW)
  dst_p = jnp.concatenate(
      [dst, N + (jnp.arange(npad, dtype=jnp.int32) % 16)]).reshape(
          _CHUNKS, _CW)

  seg = _sc_segment_sum(True)
  row = lambda v: v.reshape(1, D)
  dp = _sc_segment_sum(False)(x, src_p, dst_p)
  p1 = seg(x, src_p, dst_p)
  x1 = _layer_call(x, p1, dp, W1, row(b1), row(g1), row(be1))
  p2 = seg(x1, src_p, dst_p)
  x2 = _layer_call(x1, p2, dp, W2, row(b2), row(g2), row(be2))
  p3 = seg(x2, src_p, dst_p)
  out = _final_call(x2, p3, dp, W3, row(b3), row(g3), row(be3), x1, Wl,
                    bl.reshape(1, D))
  return out
